# baseline (speedup 1.0000x reference)
import functools

import jax
import jax.numpy as jnp
from jax import lax
from jax.experimental import pallas as pl
from jax.experimental.pallas import tpu as pltpu
from jax.experimental.pallas import tpu_sc as plsc

VOCAB = 51
EMB_DIM = 100
OUT_DIM = 51
BATCH = 4096
SEQ = 200

NC, NS = 2, 16
NW = NC * NS
L = 16
TOTAL = BATCH * SEQ
PER_W = TOTAL // NW
CHUNK = 512
NCHUNK = PER_W // CHUNK
NGROUP = CHUNK // L
TFLAT = VOCAB * OUT_DIM


def _table_body(emb_ref, w_ref, b_ref, out_ref):
    out_ref[...] = (
        jnp.dot(emb_ref[...], w_ref[...], preferred_element_type=jnp.float32)
        + b_ref[...]
    )


def _fused_table(emb_table, W, b):
    return pl.pallas_call(
        _table_body,
        out_shape=jax.ShapeDtypeStruct((VOCAB, OUT_DIM), jnp.float32),
    )(emb_table, W, b.reshape(1, OUT_DIM))


def _gather_body(table_hbm, idx_hbm, out_hbm, tab_v, idx_v, buf0, buf1,
                 sem0, sem1):
    wid = lax.axis_index("s") * NC + lax.axis_index("c")
    pltpu.sync_copy(table_hbm, tab_v)
    pltpu.sync_copy(idx_hbm.at[wid], idx_v)

    bufs = (buf0, buf1)
    sems = (sem0, sem1)

    def _fill(c, k):
        buf = bufs[k]

        def _group(g, carry):
            idxv = idx_v[c, pl.ds(g * L, L)] * OUT_DIM
            for u in range(L):
                src = idxv[u]
                dst = (g * L + u) * OUT_DIM
                for off in (0, 16, 32, 35):
                    buf[pl.ds(dst + off, L)] = tab_v[pl.ds(src + off, L)]
            return carry

        lax.fori_loop(0, NGROUP, _group, 0)

    def _out(c, k):
        pltpu.async_copy(bufs[k], out_hbm.at[wid, c], sems[k])

    def _wait_out(c, k):
        pltpu.make_async_copy(bufs[k], out_hbm.at[wid, c], sems[k]).wait()

    _fill(0, 0)
    _out(0, 0)
    _fill(1, 1)
    _out(1, 1)

    def _steady(c0):
        for k in range(2):
            c = c0 + k
            _wait_out(c - 2, k)
            _fill(c, k)
            _out(c, k)

    pl.loop(2, NCHUNK, step=2)(_steady)

    _wait_out(NCHUNK - 2, 0)
    _wait_out(NCHUNK - 1, 1)


@functools.partial(
    pl.kernel,
    mesh=plsc.VectorSubcoreMesh(core_axis_name="c", subcore_axis_name="s"),
    out_type=jax.ShapeDtypeStruct((NW, NCHUNK, CHUNK * OUT_DIM), jnp.float32),
    scratch_types=[
        pltpu.VMEM((TFLAT,), jnp.float32),
        pltpu.VMEM((NCHUNK, CHUNK), jnp.int32),
        pltpu.VMEM((CHUNK * OUT_DIM,), jnp.float32),
        pltpu.VMEM((CHUNK * OUT_DIM,), jnp.float32),
        pltpu.SemaphoreType.DMA,
        pltpu.SemaphoreType.DMA,
    ],
    compiler_params=pltpu.CompilerParams(use_tc_tiling_on_sc=False,
                                         needs_layout_passes=False),
)
def _sc_gather(table_hbm, idx_hbm, out_hbm, tab_v, idx_v, buf0, buf1,
               sem0, sem1):
    _gather_body(table_hbm, idx_hbm, out_hbm, tab_v, idx_v, buf0, buf1,
                 sem0, sem1)


def kernel(x, emb_table, W, b):
    table = _fused_table(emb_table, W, b)
    idx = x.astype(jnp.int32).reshape(NW, NCHUNK, CHUNK)
    out = _sc_gather(table.reshape(TFLAT), idx)
    return out.reshape(BATCH, SEQ, OUT_DIM)


# speedup vs baseline: 1.1679x; 1.1679x over previous
import functools

import jax
import jax.numpy as jnp
from jax import lax
from jax.experimental import pallas as pl
from jax.experimental.pallas import tpu as pltpu
from jax.experimental.pallas import tpu_sc as plsc

VOCAB = 51
EMB_DIM = 100
OUT_DIM = 51
BATCH = 4096
SEQ = 200

NC, NS = 2, 16
NW = NC * NS
L = 16
TOTAL = BATCH * SEQ
PER_W = TOTAL // NW
CHUNK = 512
NCHUNK = PER_W // CHUNK
NGROUP = CHUNK // L
TFLAT = VOCAB * OUT_DIM


def _table_body(emb_ref, w_ref, b_ref, out_ref):
    out_ref[...] = (
        jnp.dot(emb_ref[...], w_ref[...], preferred_element_type=jnp.float32)
        + b_ref[...]
    )


def _fused_table(emb_table, W, b):
    return pl.pallas_call(
        _table_body,
        out_shape=jax.ShapeDtypeStruct((VOCAB, OUT_DIM), jnp.float32),
    )(emb_table, W, b.reshape(1, OUT_DIM))


def _gather_body(table_hbm, idx_hbm, out_hbm, tab_v, idx_v, buf0, buf1,
                 sem0, sem1):
    wid = lax.axis_index("s") * NC + lax.axis_index("c")
    pltpu.sync_copy(table_hbm, tab_v)
    pltpu.sync_copy(idx_hbm.at[wid], idx_v)

    bufs = (buf0, buf1)
    sems = (sem0, sem1)
    lane = lax.iota(jnp.int32, L)
    lane51 = lane * OUT_DIM

    def _fill(c, k):
        buf = bufs[k]

        def _group(g2, carry):
            g = g2 * 2
            base_a = idx_v[c, pl.ds(g * L, L)] * OUT_DIM
            base_b = idx_v[c, pl.ds((g + 1) * L, L)] * OUT_DIM
            dst_a = g * (L * OUT_DIM) + lane51
            dst_b = (g + 1) * (L * OUT_DIM) + lane51
            for col in range(OUT_DIM):
                va = plsc.load_gather(tab_v, [base_a + col])
                vb = plsc.load_gather(tab_v, [base_b + col])
                plsc.store_scatter(buf, [dst_a + col], va)
                plsc.store_scatter(buf, [dst_b + col], vb)
            return carry

        lax.fori_loop(0, NGROUP // 2, _group, 0)

    def _out(c, k):
        pltpu.async_copy(bufs[k], out_hbm.at[wid, c], sems[k])

    def _wait_out(c, k):
        pltpu.make_async_copy(bufs[k], out_hbm.at[wid, c], sems[k]).wait()

    _fill(0, 0)
    _out(0, 0)
    _fill(1, 1)
    _out(1, 1)

    def _steady(c0):
        for k in range(2):
            c = c0 + k
            _wait_out(c - 2, k)
            _fill(c, k)
            _out(c, k)

    pl.loop(2, NCHUNK, step=2)(_steady)

    _wait_out(NCHUNK - 2, 0)
    _wait_out(NCHUNK - 1, 1)


@functools.partial(
    pl.kernel,
    mesh=plsc.VectorSubcoreMesh(core_axis_name="c", subcore_axis_name="s"),
    out_type=jax.ShapeDtypeStruct((NW, NCHUNK, CHUNK * OUT_DIM), jnp.float32),
    scratch_types=[
        pltpu.VMEM((TFLAT,), jnp.float32),
        pltpu.VMEM((NCHUNK, CHUNK), jnp.int32),
        pltpu.VMEM((CHUNK * OUT_DIM,), jnp.float32),
        pltpu.VMEM((CHUNK * OUT_DIM,), jnp.float32),
        pltpu.SemaphoreType.DMA,
        pltpu.SemaphoreType.DMA,
    ],
    compiler_params=pltpu.CompilerParams(use_tc_tiling_on_sc=False,
                                         needs_layout_passes=False),
)
def _sc_gather(table_hbm, idx_hbm, out_hbm, tab_v, idx_v, buf0, buf1,
               sem0, sem1):
    _gather_body(table_hbm, idx_hbm, out_hbm, tab_v, idx_v, buf0, buf1,
                 sem0, sem1)


def kernel(x, emb_table, W, b):
    table = _fused_table(emb_table, W, b)
    idx = x.astype(jnp.int32).reshape(NW, NCHUNK, CHUNK)
    out = _sc_gather(table.reshape(TFLAT), idx)
    return out.reshape(BATCH, SEQ, OUT_DIM)


# speedup vs baseline: 1.3008x; 1.1138x over previous
import functools

import jax
import jax.numpy as jnp
from jax import lax
from jax.experimental import pallas as pl
from jax.experimental.pallas import tpu as pltpu
from jax.experimental.pallas import tpu_sc as plsc

VOCAB = 51
EMB_DIM = 100
OUT_DIM = 51
BATCH = 4096
SEQ = 200

NC, NS = 2, 16
NW = NC * NS
L = 16
TOTAL = BATCH * SEQ
PER_W = TOTAL // NW
CHUNK = 512
NCHUNK = PER_W // CHUNK
NGROUP = CHUNK // L
TFLAT = VOCAB * OUT_DIM


def _table_body(emb_ref, w_ref, b_ref, out_ref):
    out_ref[...] = (
        jnp.dot(emb_ref[...], w_ref[...], preferred_element_type=jnp.float32)
        + b_ref[...]
    )


def _fused_table(emb_table, W, b):
    return pl.pallas_call(
        _table_body,
        out_shape=jax.ShapeDtypeStruct((VOCAB, OUT_DIM), jnp.float32),
    )(emb_table, W, b.reshape(1, OUT_DIM))


def _gather_body(table_hbm, idx_hbm, out_hbm, tab_v, idx_v, buf0, buf1,
                 sem0, sem1):
    wid = lax.axis_index("s") * NC + lax.axis_index("c")
    pltpu.sync_copy(table_hbm, tab_v)
    pltpu.sync_copy(idx_hbm.at[wid], idx_v)

    bufs = (buf0, buf1)
    sems = (sem0, sem1)
    lane = lax.iota(jnp.int32, L)
    lane51 = lane * OUT_DIM

    def _fill(c, k):
        buf = bufs[k]

        @plsc.parallel_loop(0, NGROUP // 2, unroll=2)
        def _group(g2):
            g = g2 * 2
            base_a = idx_v[c, pl.ds(g * L, L)] * OUT_DIM
            base_b = idx_v[c, pl.ds((g + 1) * L, L)] * OUT_DIM
            dst_a = g * (L * OUT_DIM) + lane51
            dst_b = (g + 1) * (L * OUT_DIM) + lane51
            for col in range(OUT_DIM):
                va = plsc.load_gather(tab_v, [base_a + col])
                vb = plsc.load_gather(tab_v, [base_b + col])
                plsc.store_scatter(buf, [dst_a + col], va)
                plsc.store_scatter(buf, [dst_b + col], vb)

    def _out(c, k):
        pltpu.async_copy(bufs[k], out_hbm.at[wid, c], sems[k])

    def _wait_out(c, k):
        pltpu.make_async_copy(bufs[k], out_hbm.at[wid, c], sems[k]).wait()

    _fill(0, 0)
    _out(0, 0)
    _fill(1, 1)
    _out(1, 1)

    def _steady(c0):
        for k in range(2):
            c = c0 + k
            _wait_out(c - 2, k)
            _fill(c, k)
            _out(c, k)

    pl.loop(2, NCHUNK, step=2)(_steady)

    _wait_out(NCHUNK - 2, 0)
    _wait_out(NCHUNK - 1, 1)


@functools.partial(
    pl.kernel,
    mesh=plsc.VectorSubcoreMesh(core_axis_name="c", subcore_axis_name="s"),
    out_type=jax.ShapeDtypeStruct((NW, NCHUNK, CHUNK * OUT_DIM), jnp.float32),
    scratch_types=[
        pltpu.VMEM((TFLAT,), jnp.float32),
        pltpu.VMEM((NCHUNK, CHUNK), jnp.int32),
        pltpu.VMEM((CHUNK * OUT_DIM,), jnp.float32),
        pltpu.VMEM((CHUNK * OUT_DIM,), jnp.float32),
        pltpu.SemaphoreType.DMA,
        pltpu.SemaphoreType.DMA,
    ],
    compiler_params=pltpu.CompilerParams(use_tc_tiling_on_sc=False,
                                         needs_layout_passes=False),
)
def _sc_gather(table_hbm, idx_hbm, out_hbm, tab_v, idx_v, buf0, buf1,
               sem0, sem1):
    _gather_body(table_hbm, idx_hbm, out_hbm, tab_v, idx_v, buf0, buf1,
                 sem0, sem1)


def kernel(x, emb_table, W, b):
    table = _fused_table(emb_table, W, b)
    idx = x.astype(jnp.int32).reshape(NW, NCHUNK, CHUNK)
    out = _sc_gather(table.reshape(TFLAT), idx)
    return out.reshape(BATCH, SEQ, OUT_DIM)
